# baseline (speedup 1.0000x reference)
import functools

import jax
import jax.numpy as jnp
from jax import lax
from jax.experimental import pallas as pl
from jax.experimental.pallas import tpu as pltpu
from jax.experimental.pallas import tpu_sc as plsc

_N = 1024
_NT = _N * _N
_E_INC = 65536
_E_EE = 32768
_E_MOT = 32768
_NSUB = 16
_SLICE = _NT // _NSUB
_ZCH = 16384


def _sc_phase(sid, maj, mnr, ept, out_slot, out, spmem, rstage, cstage,
              idx2d, ones_v, zeros_v, sem):
    nch = ept // 128
    start = sid * ept
    pltpu.sync_copy(maj.at[pl.ds(start, ept)], rstage.at[pl.ds(0, ept)])
    pltpu.sync_copy(mnr.at[pl.ds(start, ept)], cstage.at[pl.ds(0, ept)])
    for j in range(nch):
        for i in range(8):
            off = j * 128 + i * 16
            r = rstage[pl.ds(off, 16)]
            c = cstage[pl.ds(off, 16)]
            idx2d[j, pl.ds(i * 16, 16)] = r * _N + c
    for z in range(_SLICE // _ZCH):
        pltpu.sync_copy(zeros_v, spmem.at[pl.ds(sid * _SLICE + z * _ZCH, _ZCH)])
    plsc.subcore_barrier()
    copies = [pltpu.async_copy(ones_v, spmem.at[idx2d.at[j]], sem, add=True)
              for j in range(nch)]
    for cp in copies:
        cp.wait()
    plsc.subcore_barrier()
    pltpu.sync_copy(spmem.at[pl.ds(sid * _SLICE, _SLICE)],
                    out.at[out_slot, pl.ds(sid * _SLICE, _SLICE)])


def _init_consts(ones_v, zeros_v):
    for i in range(128 // 16):
        ones_v[pl.ds(i * 16, 16)] = jnp.ones((16,), jnp.float32)

    def zbody(i, carry):
        zeros_v[pl.ds(i * 16, 16)] = jnp.zeros((16,), jnp.float32)
        return carry

    lax.fori_loop(0, _ZCH // 16, zbody, 0)


def _tables_a_body(inc_r, inc_c, out,
                   spmem, rstage, cstage, idx2d, ones_v, zeros_v, sem):
    cid = lax.axis_index("c")
    sid = lax.axis_index("s")
    _init_consts(ones_v, zeros_v)

    @pl.when(cid == 0)
    def _():
        _sc_phase(sid, inc_r, inc_c, _E_INC // _NSUB, 0, out, spmem,
                  rstage, cstage, idx2d, ones_v, zeros_v, sem)

    @pl.when(cid == 1)
    def _():
        _sc_phase(sid, inc_c, inc_r, _E_INC // _NSUB, 1, out, spmem,
                  rstage, cstage, idx2d, ones_v, zeros_v, sem)


def _tables_b_body(ee_r, ee_c, mot_r, mot_c, out,
                   spmem, rstage, cstage, idx2d, ones_v, zeros_v, sem):
    cid = lax.axis_index("c")
    sid = lax.axis_index("s")
    _init_consts(ones_v, zeros_v)

    @pl.when(cid == 0)
    def _():
        _sc_phase(sid, ee_c, ee_r, _E_EE // _NSUB, 0, out, spmem,
                  rstage, cstage, idx2d, ones_v, zeros_v, sem)

    @pl.when(cid == 1)
    def _():
        _sc_phase(sid, mot_r, mot_c, _E_MOT // _NSUB, 1, out, spmem,
                  rstage, cstage, idx2d, ones_v, zeros_v, sem)


_SC_SCRATCH = lambda: [
    pltpu.VMEM_SHARED((_NT,), jnp.float32),
    pltpu.VMEM((_E_INC // _NSUB,), jnp.int32),
    pltpu.VMEM((_E_INC // _NSUB,), jnp.int32),
    pltpu.VMEM((_E_INC // _NSUB // 128, 128), jnp.int32),
    pltpu.VMEM((128,), jnp.float32),
    pltpu.VMEM((_ZCH,), jnp.float32),
    pltpu.SemaphoreType.DMA,
]


@functools.lru_cache(maxsize=1)
def _tables_sc_a():
    return pl.kernel(
        _tables_a_body,
        out_type=jax.ShapeDtypeStruct((2, _NT), jnp.float32),
        mesh=plsc.VectorSubcoreMesh(core_axis_name="c", subcore_axis_name="s"),
        scratch_types=_SC_SCRATCH(),
    )


@functools.lru_cache(maxsize=1)
def _tables_sc_b():
    return pl.kernel(
        _tables_b_body,
        out_type=jax.ShapeDtypeStruct((2, _NT), jnp.float32),
        mesh=plsc.VectorSubcoreMesh(core_axis_name="c", subcore_axis_name="s"),
        scratch_types=_SC_SCRATCH(),
    )


def _dense_body(h_ref, ht_ref, at_ref, emb_ref, hw_ref, hb_ref,
                l1w_ref, l1b_ref, gw_ref, gb_ref, l2w_ref, l2b_ref, y2_ref):
    x = jnp.dot(emb_ref[...], hw_ref[...], preferred_element_type=jnp.float32)
    hm = h_ref[...]
    htm = ht_ref[...]
    d = jnp.sum(hm, axis=1)
    bdeg = jnp.sum(htm, axis=1)
    dinv = jnp.where(d > 0, 1.0 / d, 0.0)
    binv = jnp.where(bdeg > 0, 1.0 / bdeg, 0.0)
    e = jnp.dot(htm, x, preferred_element_type=jnp.float32) * binv[:, None]
    n = jnp.dot(hm, e, preferred_element_type=jnp.float32) * dinv[:, None]
    n = n + hb_ref[...]
    y = jnp.where(n >= 0, n, 0.01 * n)
    y = jnp.dot(y, l1w_ref[...], preferred_element_type=jnp.float32)
    y = y + l1b_ref[...]
    g = lax.dot_general(y, y, (((0,), (0,)), ((), ())),
                        preferred_element_type=jnp.float32)
    g = jnp.maximum(g, 0.0)
    mean = jnp.dot(htm, g, preferred_element_type=jnp.float32)
    mean = mean * (1.0 / jnp.maximum(bdeg, 1.0))[:, None]
    xg = jnp.dot(mean, gw_ref[...], preferred_element_type=jnp.float32)
    atm = at_ref[...]
    deg = jnp.sum(atm, axis=1) + 1.0
    gdinv = lax.rsqrt(deg)
    xs = xg * gdinv[:, None]
    z = (jnp.dot(atm, xs, preferred_element_type=jnp.float32) + xs)
    z = z * gdinv[:, None] + gb_ref[...]
    z = jnp.maximum(z, 0.0)
    y2 = jnp.dot(z, l2w_ref[...], preferred_element_type=jnp.float32)
    y2_ref[...] = y2 + l2b_ref[...]


def _head_body(mhm_ref, y2_ref, l3w_ref, l3b_ref, ow_ref, ob_ref, out_ref):
    def body(k, carry):
        accm, accv = carry
        mbk = jnp.minimum(mhm_ref[pl.ds(k * 128, 128), :], 1.0)
        vbk = jnp.maximum(y2_ref[pl.ds(k * 128, 128), :], 0.0)
        mb16 = mbk.astype(jnp.bfloat16)
        vb16 = vbk.astype(jnp.bfloat16)
        for r in range(16):
            if r % 4 < 3:
                mbr = mbk[r * 8:(r + 1) * 8, :]
                vbr = vbk[r * 8:(r + 1) * 8, :]
                sel = lax.dot_general(mbr[:, :, None], vbr[:, None, :],
                                      (((2,), (1,)), ((0,), (0,))),
                                      preferred_element_type=jnp.float32)
                accm = jnp.maximum(accm, jnp.max(sel, axis=0))
            else:
                mbr = mb16[r * 8:(r + 1) * 8, :]
                vbr = vb16[r * 8:(r + 1) * 8, :]
                sel = mbr[:, :, None] * vbr[:, None, :]
                accv = jnp.maximum(accv, jnp.max(sel, axis=0))
        return accm, accv

    accm, accv = lax.fori_loop(
        0, _N // 128, body,
        (jnp.zeros((128, 512), jnp.float32), jnp.zeros((128, 512), jnp.bfloat16)))
    m = jnp.maximum(accm, accv.astype(jnp.float32))
    h = jnp.dot(m, l3w_ref[...], preferred_element_type=jnp.float32)
    h = h + l3b_ref[...]
    o = jnp.dot(h, ow_ref[...], preferred_element_type=jnp.float32)
    out_ref[...] = o + ob_ref[...]


def kernel(edge_index, edge_edge_index, motif_edge_index, node2vec_emb,
           hconv_W, hconv_b, lin1_W, lin1_b, gcn_W, gcn_b,
           lin2_W, lin2_b, eout_W, eout_b, lin3_W, lin3_b, out_W, out_b):
    t01 = _tables_sc_a()(edge_index[0], edge_index[1])
    t23 = _tables_sc_b()(edge_edge_index[0], edge_edge_index[1],
                         motif_edge_index[0], motif_edge_index[1])
    t01_2d = t01.reshape(2 * _N, _N)
    t23_2d = t23.reshape(2 * _N, _N)

    full = lambda r, c: pl.BlockSpec((r, c), lambda i: (0, 0))
    y2 = pl.pallas_call(
        _dense_body,
        grid=(1,),
        in_specs=[
            pl.BlockSpec((_N, _N), lambda i: (0, 0)),
            pl.BlockSpec((_N, _N), lambda i: (1, 0)),
            pl.BlockSpec((_N, _N), lambda i: (0, 0)),
            full(_N, _N), full(_N, _N), full(1, _N), full(_N, _N), full(1, _N),
            full(_N, 512), full(1, 512), full(512, 512), full(1, 512),
        ],
        out_specs=pl.BlockSpec((_N, 512), lambda i: (0, 0)),
        out_shape=jax.ShapeDtypeStruct((_N, 512), jnp.float32),
    )(t01_2d, t01_2d, t23_2d, node2vec_emb, hconv_W, hconv_b.reshape(1, -1),
      lin1_W, lin1_b.reshape(1, -1), gcn_W, gcn_b.reshape(1, -1),
      lin2_W, lin2_b.reshape(1, -1))

    out = pl.pallas_call(
        _head_body,
        grid=(8,),
        in_specs=[
            pl.BlockSpec((_N, 128), lambda i: (1, i)),
            pl.BlockSpec((_N, 512), lambda i: (0, 0)),
            pl.BlockSpec((512, 256), lambda i: (0, 0)),
            pl.BlockSpec((1, 256), lambda i: (0, 0)),
            pl.BlockSpec((256, 128), lambda i: (0, 0)),
            pl.BlockSpec((1, 128), lambda i: (0, 0)),
        ],
        out_specs=pl.BlockSpec((128, 128), lambda i: (i, 0)),
        out_shape=jax.ShapeDtypeStruct((_N, 128), jnp.float32),
    )(t23_2d, y2, lin3_W, lin3_b.reshape(1, -1), out_W, out_b.reshape(1, -1))
    return out


# speedup vs baseline: 1.2114x; 1.2114x over previous
import functools

import jax
import jax.numpy as jnp
from jax import lax
from jax.experimental import pallas as pl
from jax.experimental.pallas import tpu as pltpu
from jax.experimental.pallas import tpu_sc as plsc

_N = 1024
_NT = _N * _N
_E_INC = 65536
_E_EE = 32768
_E_MOT = 32768
_NSUB = 16
_SLICE = _NT // _NSUB
_ZCH = 16384


def _sc_phase(sid, maj, mnr, ept, out_slot, out, spmem, rstage, cstage,
              idx2d, ones_v, zeros_v, sem):
    nch = ept // 128
    start = sid * ept
    pltpu.sync_copy(maj.at[pl.ds(start, ept)], rstage.at[pl.ds(0, ept)])
    pltpu.sync_copy(mnr.at[pl.ds(start, ept)], cstage.at[pl.ds(0, ept)])
    for j in range(nch):
        for i in range(8):
            off = j * 128 + i * 16
            r = rstage[pl.ds(off, 16)]
            c = cstage[pl.ds(off, 16)]
            idx2d[j, pl.ds(i * 16, 16)] = r * _N + c
    for z in range(_SLICE // _ZCH):
        pltpu.sync_copy(zeros_v, spmem.at[pl.ds(sid * _SLICE + z * _ZCH, _ZCH)])
    plsc.subcore_barrier()
    copies = [pltpu.async_copy(ones_v, spmem.at[idx2d.at[j]], sem, add=True)
              for j in range(nch)]
    for cp in copies:
        cp.wait()
    plsc.subcore_barrier()
    pltpu.sync_copy(spmem.at[pl.ds(sid * _SLICE, _SLICE)],
                    out.at[out_slot, pl.ds(sid * _SLICE, _SLICE)])


def _init_consts(ones_v, zeros_v):
    for i in range(128 // 16):
        ones_v[pl.ds(i * 16, 16)] = jnp.ones((16,), jnp.float32)

    def zbody(i, carry):
        zeros_v[pl.ds(i * 16, 16)] = jnp.zeros((16,), jnp.float32)
        return carry

    lax.fori_loop(0, _ZCH // 16, zbody, 0)


def _tables_a_body(inc_r, inc_c, out,
                   spmem, rstage, cstage, idx2d, ones_v, zeros_v, sem):
    cid = lax.axis_index("c")
    sid = lax.axis_index("s")
    _init_consts(ones_v, zeros_v)

    @pl.when(cid == 0)
    def _():
        _sc_phase(sid, inc_r, inc_c, _E_INC // _NSUB, 0, out, spmem,
                  rstage, cstage, idx2d, ones_v, zeros_v, sem)

    @pl.when(cid == 1)
    def _():
        _sc_phase(sid, inc_c, inc_r, _E_INC // _NSUB, 1, out, spmem,
                  rstage, cstage, idx2d, ones_v, zeros_v, sem)


def _tables_b_body(ee_r, ee_c, mot_r, mot_c, out,
                   spmem, rstage, cstage, idx2d, ones_v, zeros_v, sem):
    cid = lax.axis_index("c")
    sid = lax.axis_index("s")
    _init_consts(ones_v, zeros_v)

    @pl.when(cid == 0)
    def _():
        _sc_phase(sid, ee_c, ee_r, _E_EE // _NSUB, 0, out, spmem,
                  rstage, cstage, idx2d, ones_v, zeros_v, sem)

    @pl.when(cid == 1)
    def _():
        _sc_phase(sid, mot_r, mot_c, _E_MOT // _NSUB, 1, out, spmem,
                  rstage, cstage, idx2d, ones_v, zeros_v, sem)


_SC_SCRATCH = lambda: [
    pltpu.VMEM_SHARED((_NT,), jnp.float32),
    pltpu.VMEM((_E_INC // _NSUB,), jnp.int32),
    pltpu.VMEM((_E_INC // _NSUB,), jnp.int32),
    pltpu.VMEM((_E_INC // _NSUB // 128, 128), jnp.int32),
    pltpu.VMEM((128,), jnp.float32),
    pltpu.VMEM((_ZCH,), jnp.float32),
    pltpu.SemaphoreType.DMA,
]


@functools.lru_cache(maxsize=1)
def _tables_sc_a():
    return pl.kernel(
        _tables_a_body,
        out_type=jax.ShapeDtypeStruct((2, _NT), jnp.float32),
        mesh=plsc.VectorSubcoreMesh(core_axis_name="c", subcore_axis_name="s"),
        scratch_types=_SC_SCRATCH(),
    )


@functools.lru_cache(maxsize=1)
def _tables_sc_b():
    return pl.kernel(
        _tables_b_body,
        out_type=jax.ShapeDtypeStruct((2, _NT), jnp.float32),
        mesh=plsc.VectorSubcoreMesh(core_axis_name="c", subcore_axis_name="s"),
        scratch_types=_SC_SCRATCH(),
    )


def _dense_body(h_ref, ht_ref, at_ref, emb_ref, hw_ref, hb_ref,
                l1w_ref, l1b_ref, gw_ref, gb_ref, l2w_ref, l2b_ref, y2_ref):
    x = jnp.dot(emb_ref[...], hw_ref[...], preferred_element_type=jnp.float32)
    hm = h_ref[...]
    htm = ht_ref[...]
    d = jnp.sum(hm, axis=1)
    bdeg = jnp.sum(htm, axis=1)
    dinv = jnp.where(d > 0, 1.0 / d, 0.0)
    binv = jnp.where(bdeg > 0, 1.0 / bdeg, 0.0)
    e = jnp.dot(htm, x, preferred_element_type=jnp.float32) * binv[:, None]
    n = jnp.dot(hm, e, preferred_element_type=jnp.float32) * dinv[:, None]
    n = n + hb_ref[...]
    y = jnp.where(n >= 0, n, 0.01 * n)
    y = jnp.dot(y, l1w_ref[...], preferred_element_type=jnp.float32)
    y = y + l1b_ref[...]
    g = lax.dot_general(y, y, (((0,), (0,)), ((), ())),
                        preferred_element_type=jnp.float32)
    g = jnp.maximum(g, 0.0)
    mean = jnp.dot(htm, g, preferred_element_type=jnp.float32)
    mean = mean * (1.0 / jnp.maximum(bdeg, 1.0))[:, None]
    xg = jnp.dot(mean, gw_ref[...], preferred_element_type=jnp.float32)
    atm = at_ref[...]
    deg = jnp.sum(atm, axis=1) + 1.0
    gdinv = lax.rsqrt(deg)
    xs = xg * gdinv[:, None]
    z = (jnp.dot(atm, xs, preferred_element_type=jnp.float32) + xs)
    z = z * gdinv[:, None] + gb_ref[...]
    z = jnp.maximum(z, 0.0)
    y2 = jnp.dot(z, l2w_ref[...], preferred_element_type=jnp.float32)
    y2_ref[...] = y2 + l2b_ref[...]


def _head_body(mhm_ref, y2_ref, l3w_ref, l3b_ref, ow_ref, ob_ref, out_ref):
    def body(k, carry):
        accm, accv = carry
        mbk = jnp.minimum(mhm_ref[pl.ds(k * 128, 128), :], 1.0)
        vbk = jnp.maximum(y2_ref[pl.ds(k * 128, 128), :], 0.0)
        mb16 = mbk.astype(jnp.bfloat16)
        vb16 = vbk.astype(jnp.bfloat16)
        for r in range(16):
            if r % 2 < 1:
                mbr = mbk[r * 8:(r + 1) * 8, :]
                vbr = vbk[r * 8:(r + 1) * 8, :]
                sel = lax.dot_general(mbr[:, :, None], vbr[:, None, :],
                                      (((2,), (1,)), ((0,), (0,))),
                                      preferred_element_type=jnp.float32)
                accm = jnp.maximum(accm, jnp.max(sel, axis=0))
            else:
                mbr = mb16[r * 8:(r + 1) * 8, :]
                vbr = vb16[r * 8:(r + 1) * 8, :]
                sel = mbr[:, :, None] * vbr[:, None, :]
                accv = jnp.maximum(accv, jnp.max(sel, axis=0))
        return accm, accv

    accm, accv = lax.fori_loop(
        0, _N // 128, body,
        (jnp.zeros((128, 512), jnp.float32), jnp.zeros((128, 512), jnp.bfloat16)))
    m = jnp.maximum(accm, accv.astype(jnp.float32))
    h = jnp.dot(m, l3w_ref[...], preferred_element_type=jnp.float32)
    h = h + l3b_ref[...]
    o = jnp.dot(h, ow_ref[...], preferred_element_type=jnp.float32)
    out_ref[...] = o + ob_ref[...]


def kernel(edge_index, edge_edge_index, motif_edge_index, node2vec_emb,
           hconv_W, hconv_b, lin1_W, lin1_b, gcn_W, gcn_b,
           lin2_W, lin2_b, eout_W, eout_b, lin3_W, lin3_b, out_W, out_b):
    t01 = _tables_sc_a()(edge_index[0], edge_index[1])
    t23 = _tables_sc_b()(edge_edge_index[0], edge_edge_index[1],
                         motif_edge_index[0], motif_edge_index[1])
    t01_2d = t01.reshape(2 * _N, _N)
    t23_2d = t23.reshape(2 * _N, _N)

    full = lambda r, c: pl.BlockSpec((r, c), lambda i: (0, 0))
    y2 = pl.pallas_call(
        _dense_body,
        grid=(1,),
        in_specs=[
            pl.BlockSpec((_N, _N), lambda i: (0, 0)),
            pl.BlockSpec((_N, _N), lambda i: (1, 0)),
            pl.BlockSpec((_N, _N), lambda i: (0, 0)),
            full(_N, _N), full(_N, _N), full(1, _N), full(_N, _N), full(1, _N),
            full(_N, 512), full(1, 512), full(512, 512), full(1, 512),
        ],
        out_specs=pl.BlockSpec((_N, 512), lambda i: (0, 0)),
        out_shape=jax.ShapeDtypeStruct((_N, 512), jnp.float32),
    )(t01_2d, t01_2d, t23_2d, node2vec_emb, hconv_W, hconv_b.reshape(1, -1),
      lin1_W, lin1_b.reshape(1, -1), gcn_W, gcn_b.reshape(1, -1),
      lin2_W, lin2_b.reshape(1, -1))

    out = pl.pallas_call(
        _head_body,
        grid=(8,),
        in_specs=[
            pl.BlockSpec((_N, 128), lambda i: (1, i)),
            pl.BlockSpec((_N, 512), lambda i: (0, 0)),
            pl.BlockSpec((512, 256), lambda i: (0, 0)),
            pl.BlockSpec((1, 256), lambda i: (0, 0)),
            pl.BlockSpec((256, 128), lambda i: (0, 0)),
            pl.BlockSpec((1, 128), lambda i: (0, 0)),
        ],
        out_specs=pl.BlockSpec((128, 128), lambda i: (i, 0)),
        out_shape=jax.ShapeDtypeStruct((_N, 128), jnp.float32),
    )(t23_2d, y2, lin3_W, lin3_b.reshape(1, -1), out_W, out_b.reshape(1, -1))
    return out


# speedup vs baseline: 1.3053x; 1.0775x over previous
import functools

import jax
import jax.numpy as jnp
from jax import lax
from jax.experimental import pallas as pl
from jax.experimental.pallas import tpu as pltpu
from jax.experimental.pallas import tpu_sc as plsc

_N = 1024
_NT = _N * _N
_E_INC = 65536
_E_EE = 32768
_E_MOT = 32768
_NSUB = 16
_SLICE = _NT // _NSUB
_ZCH = 16384


def _sc_phase(sid, maj, mnr, ept, out_slot, out, spmem, rstage, cstage,
              idx2d, ones_v, zeros_v, sem):
    nch = ept // 128
    start = sid * ept
    pltpu.sync_copy(maj.at[pl.ds(start, ept)], rstage.at[pl.ds(0, ept)])
    pltpu.sync_copy(mnr.at[pl.ds(start, ept)], cstage.at[pl.ds(0, ept)])
    for j in range(nch):
        for i in range(8):
            off = j * 128 + i * 16
            r = rstage[pl.ds(off, 16)]
            c = cstage[pl.ds(off, 16)]
            idx2d[j, pl.ds(i * 16, 16)] = r * _N + c
    for z in range(_SLICE // _ZCH):
        pltpu.sync_copy(zeros_v, spmem.at[pl.ds(sid * _SLICE + z * _ZCH, _ZCH)])
    plsc.subcore_barrier()
    copies = [pltpu.async_copy(ones_v, spmem.at[idx2d.at[j]], sem, add=True)
              for j in range(nch)]
    for cp in copies:
        cp.wait()
    plsc.subcore_barrier()
    pltpu.sync_copy(spmem.at[pl.ds(sid * _SLICE, _SLICE)],
                    out.at[out_slot, pl.ds(sid * _SLICE, _SLICE)])


def _init_consts(ones_v, zeros_v):
    for i in range(128 // 16):
        ones_v[pl.ds(i * 16, 16)] = jnp.ones((16,), jnp.float32)

    def zbody(i, carry):
        zeros_v[pl.ds(i * 16, 16)] = jnp.zeros((16,), jnp.float32)
        return carry

    lax.fori_loop(0, _ZCH // 16, zbody, 0)


def _tables_a_body(inc_r, inc_c, out,
                   spmem, rstage, cstage, idx2d, ones_v, zeros_v, sem):
    cid = lax.axis_index("c")
    sid = lax.axis_index("s")
    _init_consts(ones_v, zeros_v)

    @pl.when(cid == 0)
    def _():
        _sc_phase(sid, inc_r, inc_c, _E_INC // _NSUB, 0, out, spmem,
                  rstage, cstage, idx2d, ones_v, zeros_v, sem)

    @pl.when(cid == 1)
    def _():
        _sc_phase(sid, inc_c, inc_r, _E_INC // _NSUB, 1, out, spmem,
                  rstage, cstage, idx2d, ones_v, zeros_v, sem)


def _tables_b_body(ee_r, ee_c, mot_r, mot_c, out,
                   spmem, rstage, cstage, idx2d, ones_v, zeros_v, sem):
    cid = lax.axis_index("c")
    sid = lax.axis_index("s")
    _init_consts(ones_v, zeros_v)

    @pl.when(cid == 0)
    def _():
        _sc_phase(sid, ee_c, ee_r, _E_EE // _NSUB, 0, out, spmem,
                  rstage, cstage, idx2d, ones_v, zeros_v, sem)

    @pl.when(cid == 1)
    def _():
        _sc_phase(sid, mot_r, mot_c, _E_MOT // _NSUB, 1, out, spmem,
                  rstage, cstage, idx2d, ones_v, zeros_v, sem)


_SC_SCRATCH = lambda: [
    pltpu.VMEM_SHARED((_NT,), jnp.float32),
    pltpu.VMEM((_E_INC // _NSUB,), jnp.int32),
    pltpu.VMEM((_E_INC // _NSUB,), jnp.int32),
    pltpu.VMEM((_E_INC // _NSUB // 128, 128), jnp.int32),
    pltpu.VMEM((128,), jnp.float32),
    pltpu.VMEM((_ZCH,), jnp.float32),
    pltpu.SemaphoreType.DMA,
]


@functools.lru_cache(maxsize=1)
def _tables_sc_a():
    return pl.kernel(
        _tables_a_body,
        out_type=jax.ShapeDtypeStruct((2, _NT), jnp.float32),
        mesh=plsc.VectorSubcoreMesh(core_axis_name="c", subcore_axis_name="s"),
        scratch_types=_SC_SCRATCH(),
    )


@functools.lru_cache(maxsize=1)
def _tables_sc_b():
    return pl.kernel(
        _tables_b_body,
        out_type=jax.ShapeDtypeStruct((2, _NT), jnp.float32),
        mesh=plsc.VectorSubcoreMesh(core_axis_name="c", subcore_axis_name="s"),
        scratch_types=_SC_SCRATCH(),
    )


def _dense_body(h_ref, ht_ref, at_ref, emb_ref, hw_ref, hb_ref,
                l1w_ref, l1b_ref, gw_ref, gb_ref, l2w_ref, l2b_ref, y2_ref):
    x = jnp.dot(emb_ref[...], hw_ref[...], preferred_element_type=jnp.float32)
    hm = h_ref[...]
    htm = ht_ref[...]
    d = jnp.sum(hm, axis=1)
    bdeg = jnp.sum(htm, axis=1)
    dinv = jnp.where(d > 0, 1.0 / d, 0.0)
    binv = jnp.where(bdeg > 0, 1.0 / bdeg, 0.0)
    e = jnp.dot(htm, x, preferred_element_type=jnp.float32) * binv[:, None]
    n = jnp.dot(hm, e, preferred_element_type=jnp.float32) * dinv[:, None]
    n = n + hb_ref[...]
    y = jnp.where(n >= 0, n, 0.01 * n)
    y = jnp.dot(y, l1w_ref[...], preferred_element_type=jnp.float32)
    y = y + l1b_ref[...]
    g = lax.dot_general(y, y, (((0,), (0,)), ((), ())),
                        preferred_element_type=jnp.float32)
    g = jnp.maximum(g, 0.0)
    mean = jnp.dot(htm, g, preferred_element_type=jnp.float32)
    mean = mean * (1.0 / jnp.maximum(bdeg, 1.0))[:, None]
    xg = jnp.dot(mean, gw_ref[...], preferred_element_type=jnp.float32)
    atm = at_ref[...]
    deg = jnp.sum(atm, axis=1) + 1.0
    gdinv = lax.rsqrt(deg)
    xs = xg * gdinv[:, None]
    z = (jnp.dot(atm, xs, preferred_element_type=jnp.float32) + xs)
    z = z * gdinv[:, None] + gb_ref[...]
    z = jnp.maximum(z, 0.0)
    y2 = jnp.dot(z, l2w_ref[...], preferred_element_type=jnp.float32)
    y2_ref[...] = y2 + l2b_ref[...]


def _head_body(mhm_ref, y2_ref, l3w_ref, l3b_ref, ow_ref, ob_ref, out_ref):
    def body(k, carry):
        accm, accv = carry
        mbk = jnp.minimum(mhm_ref[pl.ds(k * 128, 128), :], 1.0)
        vbk = jnp.maximum(y2_ref[pl.ds(k * 128, 128), :], 0.0)
        mb16 = mbk.astype(jnp.bfloat16)
        vb16 = vbk.astype(jnp.bfloat16)
        for r in range(16):
            if r % 8 < 3:
                mbr = mbk[r * 8:(r + 1) * 8, :]
                vbr = vbk[r * 8:(r + 1) * 8, :]
                sel = lax.dot_general(mbr[:, :, None], vbr[:, None, :],
                                      (((2,), (1,)), ((0,), (0,))),
                                      preferred_element_type=jnp.float32)
                accm = jnp.maximum(accm, jnp.max(sel, axis=0))
            else:
                mbr = mb16[r * 8:(r + 1) * 8, :]
                vbr = vb16[r * 8:(r + 1) * 8, :]
                sel = mbr[:, :, None] * vbr[:, None, :]
                accv = jnp.maximum(accv, jnp.max(sel, axis=0))
        return accm, accv

    accm, accv = lax.fori_loop(
        0, _N // 128, body,
        (jnp.zeros((128, 512), jnp.float32), jnp.zeros((128, 512), jnp.bfloat16)))
    m = jnp.maximum(accm, accv.astype(jnp.float32))
    h = jnp.dot(m, l3w_ref[...], preferred_element_type=jnp.float32)
    h = h + l3b_ref[...]
    o = jnp.dot(h, ow_ref[...], preferred_element_type=jnp.float32)
    out_ref[...] = o + ob_ref[...]


def kernel(edge_index, edge_edge_index, motif_edge_index, node2vec_emb,
           hconv_W, hconv_b, lin1_W, lin1_b, gcn_W, gcn_b,
           lin2_W, lin2_b, eout_W, eout_b, lin3_W, lin3_b, out_W, out_b):
    t01 = _tables_sc_a()(edge_index[0], edge_index[1])
    t23 = _tables_sc_b()(edge_edge_index[0], edge_edge_index[1],
                         motif_edge_index[0], motif_edge_index[1])
    t01_2d = t01.reshape(2 * _N, _N)
    t23_2d = t23.reshape(2 * _N, _N)

    full = lambda r, c: pl.BlockSpec((r, c), lambda i: (0, 0))
    y2 = pl.pallas_call(
        _dense_body,
        grid=(1,),
        in_specs=[
            pl.BlockSpec((_N, _N), lambda i: (0, 0)),
            pl.BlockSpec((_N, _N), lambda i: (1, 0)),
            pl.BlockSpec((_N, _N), lambda i: (0, 0)),
            full(_N, _N), full(_N, _N), full(1, _N), full(_N, _N), full(1, _N),
            full(_N, 512), full(1, 512), full(512, 512), full(1, 512),
        ],
        out_specs=pl.BlockSpec((_N, 512), lambda i: (0, 0)),
        out_shape=jax.ShapeDtypeStruct((_N, 512), jnp.float32),
    )(t01_2d, t01_2d, t23_2d, node2vec_emb, hconv_W, hconv_b.reshape(1, -1),
      lin1_W, lin1_b.reshape(1, -1), gcn_W, gcn_b.reshape(1, -1),
      lin2_W, lin2_b.reshape(1, -1))

    out = pl.pallas_call(
        _head_body,
        grid=(8,),
        in_specs=[
            pl.BlockSpec((_N, 128), lambda i: (1, i)),
            pl.BlockSpec((_N, 512), lambda i: (0, 0)),
            pl.BlockSpec((512, 256), lambda i: (0, 0)),
            pl.BlockSpec((1, 256), lambda i: (0, 0)),
            pl.BlockSpec((256, 128), lambda i: (0, 0)),
            pl.BlockSpec((1, 128), lambda i: (0, 0)),
        ],
        out_specs=pl.BlockSpec((128, 128), lambda i: (i, 0)),
        out_shape=jax.ShapeDtypeStruct((_N, 128), jnp.float32),
    )(t23_2d, y2, lin3_W, lin3_b.reshape(1, -1), out_W, out_b.reshape(1, -1))
    return out


# speedup vs baseline: 1.3889x; 1.0641x over previous
import functools

import jax
import jax.numpy as jnp
from jax import lax
from jax.experimental import pallas as pl
from jax.experimental.pallas import tpu as pltpu
from jax.experimental.pallas import tpu_sc as plsc

_N = 1024
_NT = _N * _N
_E_INC = 65536
_E_EE = 32768
_E_MOT = 32768
_NSUB = 16
_SLICE = _NT // _NSUB
_ZCH = 16384


def _sc_phase(sid, maj, mnr, ept, out_slot, out, spmem, rstage, cstage,
              idx2d, ones_v, zeros_v, sem):
    nch = ept // 128
    start = sid * ept
    pltpu.sync_copy(maj.at[pl.ds(start, ept)], rstage.at[pl.ds(0, ept)])
    pltpu.sync_copy(mnr.at[pl.ds(start, ept)], cstage.at[pl.ds(0, ept)])
    for j in range(nch):
        for i in range(8):
            off = j * 128 + i * 16
            r = rstage[pl.ds(off, 16)]
            c = cstage[pl.ds(off, 16)]
            idx2d[j, pl.ds(i * 16, 16)] = r * _N + c
    for z in range(_SLICE // _ZCH):
        pltpu.sync_copy(zeros_v, spmem.at[pl.ds(sid * _SLICE + z * _ZCH, _ZCH)])
    plsc.subcore_barrier()
    copies = [pltpu.async_copy(ones_v, spmem.at[idx2d.at[j]], sem, add=True)
              for j in range(nch)]
    for cp in copies:
        cp.wait()
    plsc.subcore_barrier()
    pltpu.sync_copy(spmem.at[pl.ds(sid * _SLICE, _SLICE)],
                    out.at[out_slot, pl.ds(sid * _SLICE, _SLICE)])


def _init_consts(ones_v, zeros_v):
    for i in range(128 // 16):
        ones_v[pl.ds(i * 16, 16)] = jnp.ones((16,), jnp.float32)

    def zbody(i, carry):
        zeros_v[pl.ds(i * 16, 16)] = jnp.zeros((16,), jnp.float32)
        return carry

    lax.fori_loop(0, _ZCH // 16, zbody, 0)


def _tables_a_body(inc_r, inc_c, out,
                   spmem, rstage, cstage, idx2d, ones_v, zeros_v, sem):
    cid = lax.axis_index("c")
    sid = lax.axis_index("s")
    _init_consts(ones_v, zeros_v)

    @pl.when(cid == 0)
    def _():
        _sc_phase(sid, inc_r, inc_c, _E_INC // _NSUB, 0, out, spmem,
                  rstage, cstage, idx2d, ones_v, zeros_v, sem)

    @pl.when(cid == 1)
    def _():
        _sc_phase(sid, inc_c, inc_r, _E_INC // _NSUB, 1, out, spmem,
                  rstage, cstage, idx2d, ones_v, zeros_v, sem)


def _tables_b_body(ee_r, ee_c, mot_r, mot_c, out,
                   spmem, rstage, cstage, idx2d, ones_v, zeros_v, sem):
    cid = lax.axis_index("c")
    sid = lax.axis_index("s")
    _init_consts(ones_v, zeros_v)

    @pl.when(cid == 0)
    def _():
        _sc_phase(sid, ee_c, ee_r, _E_EE // _NSUB, 0, out, spmem,
                  rstage, cstage, idx2d, ones_v, zeros_v, sem)

    @pl.when(cid == 1)
    def _():
        _sc_phase(sid, mot_r, mot_c, _E_MOT // _NSUB, 1, out, spmem,
                  rstage, cstage, idx2d, ones_v, zeros_v, sem)


_SC_SCRATCH = lambda: [
    pltpu.VMEM_SHARED((_NT,), jnp.float32),
    pltpu.VMEM((_E_INC // _NSUB,), jnp.int32),
    pltpu.VMEM((_E_INC // _NSUB,), jnp.int32),
    pltpu.VMEM((_E_INC // _NSUB // 128, 128), jnp.int32),
    pltpu.VMEM((128,), jnp.float32),
    pltpu.VMEM((_ZCH,), jnp.float32),
    pltpu.SemaphoreType.DMA,
]


@functools.lru_cache(maxsize=1)
def _tables_sc_a():
    return pl.kernel(
        _tables_a_body,
        out_type=jax.ShapeDtypeStruct((2, _NT), jnp.float32),
        mesh=plsc.VectorSubcoreMesh(core_axis_name="c", subcore_axis_name="s"),
        scratch_types=_SC_SCRATCH(),
    )


@functools.lru_cache(maxsize=1)
def _tables_sc_b():
    return pl.kernel(
        _tables_b_body,
        out_type=jax.ShapeDtypeStruct((2, _NT), jnp.float32),
        mesh=plsc.VectorSubcoreMesh(core_axis_name="c", subcore_axis_name="s"),
        scratch_types=_SC_SCRATCH(),
    )


def _dense_body(h_ref, ht_ref, at_ref, emb_ref, hw_ref, hb_ref,
                l1w_ref, l1b_ref, gw_ref, gb_ref, l2w_ref, l2b_ref, y2_ref):
    x = jnp.dot(emb_ref[...], hw_ref[...], preferred_element_type=jnp.float32)
    hm = h_ref[...]
    htm = ht_ref[...]
    d = jnp.sum(hm, axis=1)
    bdeg = jnp.sum(htm, axis=1)
    dinv = jnp.where(d > 0, 1.0 / d, 0.0)
    binv = jnp.where(bdeg > 0, 1.0 / bdeg, 0.0)
    e = jnp.dot(htm, x, preferred_element_type=jnp.float32) * binv[:, None]
    n = jnp.dot(hm, e, preferred_element_type=jnp.float32) * dinv[:, None]
    n = n + hb_ref[...]
    y = jnp.where(n >= 0, n, 0.01 * n)
    y = jnp.dot(y, l1w_ref[...], preferred_element_type=jnp.float32)
    y = y + l1b_ref[...]
    g = lax.dot_general(y, y, (((0,), (0,)), ((), ())),
                        preferred_element_type=jnp.float32)
    g = jnp.maximum(g, 0.0)
    mean = jnp.dot(htm, g, preferred_element_type=jnp.float32)
    mean = mean * (1.0 / jnp.maximum(bdeg, 1.0))[:, None]
    xg = jnp.dot(mean, gw_ref[...], preferred_element_type=jnp.float32)
    atm = at_ref[...]
    deg = jnp.sum(atm, axis=1) + 1.0
    gdinv = lax.rsqrt(deg)
    xs = xg * gdinv[:, None]
    z = (jnp.dot(atm, xs, preferred_element_type=jnp.float32) + xs)
    z = z * gdinv[:, None] + gb_ref[...]
    z = jnp.maximum(z, 0.0)
    y2 = jnp.dot(z, l2w_ref[...], preferred_element_type=jnp.float32)
    y2_ref[...] = y2 + l2b_ref[...]


def _head_body(mhm_ref, y2_ref, l3w_ref, l3b_ref, ow_ref, ob_ref, out_ref):
    def body(k, carry):
        accm, accv = carry
        mbk = jnp.minimum(mhm_ref[pl.ds(k * 128, 128), :], 1.0)
        vbk = jnp.maximum(y2_ref[pl.ds(k * 128, 128), :], 0.0)
        mb16 = mbk.astype(jnp.bfloat16)
        vb16 = vbk.astype(jnp.bfloat16)
        for r in range(16):
            if r % 4 < 1:
                mbr = mbk[r * 8:(r + 1) * 8, :]
                vbr = vbk[r * 8:(r + 1) * 8, :]
                sel = lax.dot_general(mbr[:, :, None], vbr[:, None, :],
                                      (((2,), (1,)), ((0,), (0,))),
                                      preferred_element_type=jnp.float32)
                accm = jnp.maximum(accm, jnp.max(sel, axis=0))
            else:
                mbr = mb16[r * 8:(r + 1) * 8, :]
                vbr = vb16[r * 8:(r + 1) * 8, :]
                sel = mbr[:, :, None] * vbr[:, None, :]
                accv = jnp.maximum(accv, jnp.max(sel, axis=0))
        return accm, accv

    accm, accv = lax.fori_loop(
        0, _N // 128, body,
        (jnp.zeros((128, 512), jnp.float32), jnp.zeros((128, 512), jnp.bfloat16)))
    m = jnp.maximum(accm, accv.astype(jnp.float32))
    h = jnp.dot(m, l3w_ref[...], preferred_element_type=jnp.float32)
    h = h + l3b_ref[...]
    o = jnp.dot(h, ow_ref[...], preferred_element_type=jnp.float32)
    out_ref[...] = o + ob_ref[...]


def kernel(edge_index, edge_edge_index, motif_edge_index, node2vec_emb,
           hconv_W, hconv_b, lin1_W, lin1_b, gcn_W, gcn_b,
           lin2_W, lin2_b, eout_W, eout_b, lin3_W, lin3_b, out_W, out_b):
    t01 = _tables_sc_a()(edge_index[0], edge_index[1])
    t23 = _tables_sc_b()(edge_edge_index[0], edge_edge_index[1],
                         motif_edge_index[0], motif_edge_index[1])
    t01_2d = t01.reshape(2 * _N, _N)
    t23_2d = t23.reshape(2 * _N, _N)

    full = lambda r, c: pl.BlockSpec((r, c), lambda i: (0, 0))
    y2 = pl.pallas_call(
        _dense_body,
        grid=(1,),
        in_specs=[
            pl.BlockSpec((_N, _N), lambda i: (0, 0)),
            pl.BlockSpec((_N, _N), lambda i: (1, 0)),
            pl.BlockSpec((_N, _N), lambda i: (0, 0)),
            full(_N, _N), full(_N, _N), full(1, _N), full(_N, _N), full(1, _N),
            full(_N, 512), full(1, 512), full(512, 512), full(1, 512),
        ],
        out_specs=pl.BlockSpec((_N, 512), lambda i: (0, 0)),
        out_shape=jax.ShapeDtypeStruct((_N, 512), jnp.float32),
    )(t01_2d, t01_2d, t23_2d, node2vec_emb, hconv_W, hconv_b.reshape(1, -1),
      lin1_W, lin1_b.reshape(1, -1), gcn_W, gcn_b.reshape(1, -1),
      lin2_W, lin2_b.reshape(1, -1))

    out = pl.pallas_call(
        _head_body,
        grid=(8,),
        in_specs=[
            pl.BlockSpec((_N, 128), lambda i: (1, i)),
            pl.BlockSpec((_N, 512), lambda i: (0, 0)),
            pl.BlockSpec((512, 256), lambda i: (0, 0)),
            pl.BlockSpec((1, 256), lambda i: (0, 0)),
            pl.BlockSpec((256, 128), lambda i: (0, 0)),
            pl.BlockSpec((1, 128), lambda i: (0, 0)),
        ],
        out_specs=pl.BlockSpec((128, 128), lambda i: (i, 0)),
        out_shape=jax.ShapeDtypeStruct((_N, 128), jnp.float32),
    )(t23_2d, y2, lin3_W, lin3_b.reshape(1, -1), out_W, out_b.reshape(1, -1))
    return out


# speedup vs baseline: 1.4975x; 1.0782x over previous
import functools

import jax
import jax.numpy as jnp
from jax import lax
from jax.experimental import pallas as pl
from jax.experimental.pallas import tpu as pltpu
from jax.experimental.pallas import tpu_sc as plsc

_N = 1024
_NT = _N * _N
_E_INC = 65536
_E_EE = 32768
_E_MOT = 32768
_NSUB = 16
_SLICE = _NT // _NSUB
_ZCH = 16384


def _sc_phase(sid, maj, mnr, ept, out_slot, out, spmem, rstage, cstage,
              idx2d, ones_v, zeros_v, sem):
    nch = ept // 128
    start = sid * ept
    pltpu.sync_copy(maj.at[pl.ds(start, ept)], rstage.at[pl.ds(0, ept)])
    pltpu.sync_copy(mnr.at[pl.ds(start, ept)], cstage.at[pl.ds(0, ept)])
    for j in range(nch):
        for i in range(8):
            off = j * 128 + i * 16
            r = rstage[pl.ds(off, 16)]
            c = cstage[pl.ds(off, 16)]
            idx2d[j, pl.ds(i * 16, 16)] = r * _N + c
    for z in range(_SLICE // _ZCH):
        pltpu.sync_copy(zeros_v, spmem.at[pl.ds(sid * _SLICE + z * _ZCH, _ZCH)])
    plsc.subcore_barrier()
    copies = [pltpu.async_copy(ones_v, spmem.at[idx2d.at[j]], sem, add=True)
              for j in range(nch)]
    for cp in copies:
        cp.wait()
    plsc.subcore_barrier()
    pltpu.sync_copy(spmem.at[pl.ds(sid * _SLICE, _SLICE)],
                    out.at[out_slot, pl.ds(sid * _SLICE, _SLICE)])


def _init_consts(ones_v, zeros_v):
    for i in range(128 // 16):
        ones_v[pl.ds(i * 16, 16)] = jnp.ones((16,), jnp.float32)

    def zbody(i, carry):
        zeros_v[pl.ds(i * 16, 16)] = jnp.zeros((16,), jnp.float32)
        return carry

    lax.fori_loop(0, _ZCH // 16, zbody, 0)


def _tables_a_body(inc_r, inc_c, out,
                   spmem, rstage, cstage, idx2d, ones_v, zeros_v, sem):
    cid = lax.axis_index("c")
    sid = lax.axis_index("s")
    _init_consts(ones_v, zeros_v)

    @pl.when(cid == 0)
    def _():
        _sc_phase(sid, inc_r, inc_c, _E_INC // _NSUB, 0, out, spmem,
                  rstage, cstage, idx2d, ones_v, zeros_v, sem)

    @pl.when(cid == 1)
    def _():
        _sc_phase(sid, inc_c, inc_r, _E_INC // _NSUB, 1, out, spmem,
                  rstage, cstage, idx2d, ones_v, zeros_v, sem)


def _tables_b_body(ee_r, ee_c, mot_r, mot_c, out,
                   spmem, rstage, cstage, idx2d, ones_v, zeros_v, sem):
    cid = lax.axis_index("c")
    sid = lax.axis_index("s")
    _init_consts(ones_v, zeros_v)

    @pl.when(cid == 0)
    def _():
        _sc_phase(sid, ee_c, ee_r, _E_EE // _NSUB, 0, out, spmem,
                  rstage, cstage, idx2d, ones_v, zeros_v, sem)

    @pl.when(cid == 1)
    def _():
        _sc_phase(sid, mot_r, mot_c, _E_MOT // _NSUB, 1, out, spmem,
                  rstage, cstage, idx2d, ones_v, zeros_v, sem)


_SC_SCRATCH = lambda: [
    pltpu.VMEM_SHARED((_NT,), jnp.float32),
    pltpu.VMEM((_E_INC // _NSUB,), jnp.int32),
    pltpu.VMEM((_E_INC // _NSUB,), jnp.int32),
    pltpu.VMEM((_E_INC // _NSUB // 128, 128), jnp.int32),
    pltpu.VMEM((128,), jnp.float32),
    pltpu.VMEM((_ZCH,), jnp.float32),
    pltpu.SemaphoreType.DMA,
]


@functools.lru_cache(maxsize=1)
def _tables_sc_a():
    return pl.kernel(
        _tables_a_body,
        out_type=jax.ShapeDtypeStruct((2, _NT), jnp.float32),
        mesh=plsc.VectorSubcoreMesh(core_axis_name="c", subcore_axis_name="s"),
        scratch_types=_SC_SCRATCH(),
    )


@functools.lru_cache(maxsize=1)
def _tables_sc_b():
    return pl.kernel(
        _tables_b_body,
        out_type=jax.ShapeDtypeStruct((2, _NT), jnp.float32),
        mesh=plsc.VectorSubcoreMesh(core_axis_name="c", subcore_axis_name="s"),
        scratch_types=_SC_SCRATCH(),
    )


def _dense_body(h_ref, ht_ref, at_ref, emb_ref, hw_ref, hb_ref,
                l1w_ref, l1b_ref, gw_ref, gb_ref, l2w_ref, l2b_ref, y2_ref):
    x = jnp.dot(emb_ref[...], hw_ref[...], preferred_element_type=jnp.float32)
    hm = h_ref[...]
    htm = ht_ref[...]
    d = jnp.sum(hm, axis=1)
    bdeg = jnp.sum(htm, axis=1)
    dinv = jnp.where(d > 0, 1.0 / d, 0.0)
    binv = jnp.where(bdeg > 0, 1.0 / bdeg, 0.0)
    e = jnp.dot(htm, x, preferred_element_type=jnp.float32) * binv[:, None]
    n = jnp.dot(hm, e, preferred_element_type=jnp.float32) * dinv[:, None]
    n = n + hb_ref[...]
    y = jnp.where(n >= 0, n, 0.01 * n)
    y = jnp.dot(y, l1w_ref[...], preferred_element_type=jnp.float32)
    y = y + l1b_ref[...]
    g = lax.dot_general(y, y, (((0,), (0,)), ((), ())),
                        preferred_element_type=jnp.float32)
    g = jnp.maximum(g, 0.0)
    mean = jnp.dot(htm, g, preferred_element_type=jnp.float32)
    mean = mean * (1.0 / jnp.maximum(bdeg, 1.0))[:, None]
    xg = jnp.dot(mean, gw_ref[...], preferred_element_type=jnp.float32)
    atm = at_ref[...]
    deg = jnp.sum(atm, axis=1) + 1.0
    gdinv = lax.rsqrt(deg)
    xs = xg * gdinv[:, None]
    z = (jnp.dot(atm, xs, preferred_element_type=jnp.float32) + xs)
    z = z * gdinv[:, None] + gb_ref[...]
    z = jnp.maximum(z, 0.0)
    y2 = jnp.dot(z, l2w_ref[...], preferred_element_type=jnp.float32)
    y2_ref[...] = y2 + l2b_ref[...]


def _head_body(mhm_ref, y2_ref, l3w_ref, l3b_ref, ow_ref, ob_ref, out_ref):
    def body(k, carry):
        accm, accv = carry
        mbk = jnp.minimum(mhm_ref[pl.ds(k * 128, 128), :], 1.0)
        vbk = jnp.maximum(y2_ref[pl.ds(k * 128, 128), :], 0.0)
        mb16 = mbk.astype(jnp.bfloat16)
        vb16 = vbk.astype(jnp.bfloat16)
        for r in range(16):
            if r % 8 < 1:
                mbr = mbk[r * 8:(r + 1) * 8, :]
                vbr = vbk[r * 8:(r + 1) * 8, :]
                sel = lax.dot_general(mbr[:, :, None], vbr[:, None, :],
                                      (((2,), (1,)), ((0,), (0,))),
                                      preferred_element_type=jnp.float32)
                accm = jnp.maximum(accm, jnp.max(sel, axis=0))
            else:
                mbr = mb16[r * 8:(r + 1) * 8, :]
                vbr = vb16[r * 8:(r + 1) * 8, :]
                sel = mbr[:, :, None] * vbr[:, None, :]
                accv = jnp.maximum(accv, jnp.max(sel, axis=0))
        return accm, accv

    accm, accv = lax.fori_loop(
        0, _N // 128, body,
        (jnp.zeros((128, 512), jnp.float32), jnp.zeros((128, 512), jnp.bfloat16)))
    m = jnp.maximum(accm, accv.astype(jnp.float32))
    h = jnp.dot(m, l3w_ref[...], preferred_element_type=jnp.float32)
    h = h + l3b_ref[...]
    o = jnp.dot(h, ow_ref[...], preferred_element_type=jnp.float32)
    out_ref[...] = o + ob_ref[...]


def kernel(edge_index, edge_edge_index, motif_edge_index, node2vec_emb,
           hconv_W, hconv_b, lin1_W, lin1_b, gcn_W, gcn_b,
           lin2_W, lin2_b, eout_W, eout_b, lin3_W, lin3_b, out_W, out_b):
    t01 = _tables_sc_a()(edge_index[0], edge_index[1])
    t23 = _tables_sc_b()(edge_edge_index[0], edge_edge_index[1],
                         motif_edge_index[0], motif_edge_index[1])
    t01_2d = t01.reshape(2 * _N, _N)
    t23_2d = t23.reshape(2 * _N, _N)

    full = lambda r, c: pl.BlockSpec((r, c), lambda i: (0, 0))
    y2 = pl.pallas_call(
        _dense_body,
        grid=(1,),
        in_specs=[
            pl.BlockSpec((_N, _N), lambda i: (0, 0)),
            pl.BlockSpec((_N, _N), lambda i: (1, 0)),
            pl.BlockSpec((_N, _N), lambda i: (0, 0)),
            full(_N, _N), full(_N, _N), full(1, _N), full(_N, _N), full(1, _N),
            full(_N, 512), full(1, 512), full(512, 512), full(1, 512),
        ],
        out_specs=pl.BlockSpec((_N, 512), lambda i: (0, 0)),
        out_shape=jax.ShapeDtypeStruct((_N, 512), jnp.float32),
    )(t01_2d, t01_2d, t23_2d, node2vec_emb, hconv_W, hconv_b.reshape(1, -1),
      lin1_W, lin1_b.reshape(1, -1), gcn_W, gcn_b.reshape(1, -1),
      lin2_W, lin2_b.reshape(1, -1))

    out = pl.pallas_call(
        _head_body,
        grid=(8,),
        in_specs=[
            pl.BlockSpec((_N, 128), lambda i: (1, i)),
            pl.BlockSpec((_N, 512), lambda i: (0, 0)),
            pl.BlockSpec((512, 256), lambda i: (0, 0)),
            pl.BlockSpec((1, 256), lambda i: (0, 0)),
            pl.BlockSpec((256, 128), lambda i: (0, 0)),
            pl.BlockSpec((1, 128), lambda i: (0, 0)),
        ],
        out_specs=pl.BlockSpec((128, 128), lambda i: (i, 0)),
        out_shape=jax.ShapeDtypeStruct((_N, 128), jnp.float32),
    )(t23_2d, y2, lin3_W, lin3_b.reshape(1, -1), out_W, out_b.reshape(1, -1))
    return out


# speedup vs baseline: 1.7415x; 1.1630x over previous
import functools

import jax
import jax.numpy as jnp
from jax import lax
from jax.experimental import pallas as pl
from jax.experimental.pallas import tpu as pltpu
from jax.experimental.pallas import tpu_sc as plsc

_N = 1024
_NT = _N * _N
_E_INC = 65536
_E_EE = 32768
_E_MOT = 32768
_NSUB = 16
_SLICE = _NT // _NSUB
_ZCH = 16384


def _sc_phase(sid, maj, mnr, ept, out_slot, out, spmem, rstage, cstage,
              idx2d, ones_v, zeros_v, sem):
    nch = ept // 128
    start = sid * ept
    pltpu.sync_copy(maj.at[pl.ds(start, ept)], rstage.at[pl.ds(0, ept)])
    pltpu.sync_copy(mnr.at[pl.ds(start, ept)], cstage.at[pl.ds(0, ept)])
    for j in range(nch):
        for i in range(8):
            off = j * 128 + i * 16
            r = rstage[pl.ds(off, 16)]
            c = cstage[pl.ds(off, 16)]
            idx2d[j, pl.ds(i * 16, 16)] = r * _N + c
    for z in range(_SLICE // _ZCH):
        pltpu.sync_copy(zeros_v, spmem.at[pl.ds(sid * _SLICE + z * _ZCH, _ZCH)])
    plsc.subcore_barrier()
    copies = [pltpu.async_copy(ones_v, spmem.at[idx2d.at[j]], sem, add=True)
              for j in range(nch)]
    for cp in copies:
        cp.wait()
    plsc.subcore_barrier()
    pltpu.sync_copy(spmem.at[pl.ds(sid * _SLICE, _SLICE)],
                    out.at[out_slot, pl.ds(sid * _SLICE, _SLICE)])


def _init_consts(ones_v, zeros_v):
    for i in range(128 // 16):
        ones_v[pl.ds(i * 16, 16)] = jnp.ones((16,), jnp.float32)

    def zbody(i, carry):
        zeros_v[pl.ds(i * 16, 16)] = jnp.zeros((16,), jnp.float32)
        return carry

    lax.fori_loop(0, _ZCH // 16, zbody, 0)


def _tables_a_body(inc_r, inc_c, out,
                   spmem, rstage, cstage, idx2d, ones_v, zeros_v, sem):
    cid = lax.axis_index("c")
    sid = lax.axis_index("s")
    _init_consts(ones_v, zeros_v)

    @pl.when(cid == 0)
    def _():
        _sc_phase(sid, inc_r, inc_c, _E_INC // _NSUB, 0, out, spmem,
                  rstage, cstage, idx2d, ones_v, zeros_v, sem)

    @pl.when(cid == 1)
    def _():
        _sc_phase(sid, inc_c, inc_r, _E_INC // _NSUB, 1, out, spmem,
                  rstage, cstage, idx2d, ones_v, zeros_v, sem)


def _tables_b_body(ee_r, ee_c, mot_r, mot_c, out,
                   spmem, rstage, cstage, idx2d, ones_v, zeros_v, sem):
    cid = lax.axis_index("c")
    sid = lax.axis_index("s")
    _init_consts(ones_v, zeros_v)

    @pl.when(cid == 0)
    def _():
        _sc_phase(sid, ee_c, ee_r, _E_EE // _NSUB, 0, out, spmem,
                  rstage, cstage, idx2d, ones_v, zeros_v, sem)

    @pl.when(cid == 1)
    def _():
        _sc_phase(sid, mot_r, mot_c, _E_MOT // _NSUB, 1, out, spmem,
                  rstage, cstage, idx2d, ones_v, zeros_v, sem)


_SC_SCRATCH = lambda: [
    pltpu.VMEM_SHARED((_NT,), jnp.float32),
    pltpu.VMEM((_E_INC // _NSUB,), jnp.int32),
    pltpu.VMEM((_E_INC // _NSUB,), jnp.int32),
    pltpu.VMEM((_E_INC // _NSUB // 128, 128), jnp.int32),
    pltpu.VMEM((128,), jnp.float32),
    pltpu.VMEM((_ZCH,), jnp.float32),
    pltpu.SemaphoreType.DMA,
]


@functools.lru_cache(maxsize=1)
def _tables_sc_a():
    return pl.kernel(
        _tables_a_body,
        out_type=jax.ShapeDtypeStruct((2, _NT), jnp.float32),
        mesh=plsc.VectorSubcoreMesh(core_axis_name="c", subcore_axis_name="s"),
        scratch_types=_SC_SCRATCH(),
    )


@functools.lru_cache(maxsize=1)
def _tables_sc_b():
    return pl.kernel(
        _tables_b_body,
        out_type=jax.ShapeDtypeStruct((2, _NT), jnp.float32),
        mesh=plsc.VectorSubcoreMesh(core_axis_name="c", subcore_axis_name="s"),
        scratch_types=_SC_SCRATCH(),
    )


def _dense_body(h_ref, ht_ref, at_ref, emb_ref, hw_ref, hb_ref,
                l1w_ref, l1b_ref, gw_ref, gb_ref, l2w_ref, l2b_ref, y2_ref):
    x = jnp.dot(emb_ref[...], hw_ref[...], preferred_element_type=jnp.float32)
    hm = h_ref[...]
    htm = ht_ref[...]
    d = jnp.sum(hm, axis=1)
    bdeg = jnp.sum(htm, axis=1)
    dinv = jnp.where(d > 0, 1.0 / d, 0.0)
    binv = jnp.where(bdeg > 0, 1.0 / bdeg, 0.0)
    e = jnp.dot(htm, x, preferred_element_type=jnp.float32) * binv[:, None]
    n = jnp.dot(hm, e, preferred_element_type=jnp.float32) * dinv[:, None]
    n = n + hb_ref[...]
    y = jnp.where(n >= 0, n, 0.01 * n)
    y = jnp.dot(y, l1w_ref[...], preferred_element_type=jnp.float32)
    y = y + l1b_ref[...]
    g = lax.dot_general(y, y, (((0,), (0,)), ((), ())),
                        preferred_element_type=jnp.float32)
    g = jnp.maximum(g, 0.0)
    mean = jnp.dot(htm, g, preferred_element_type=jnp.float32)
    mean = mean * (1.0 / jnp.maximum(bdeg, 1.0))[:, None]
    xg = jnp.dot(mean, gw_ref[...], preferred_element_type=jnp.float32)
    atm = at_ref[...]
    deg = jnp.sum(atm, axis=1) + 1.0
    gdinv = lax.rsqrt(deg)
    xs = xg * gdinv[:, None]
    z = (jnp.dot(atm, xs, preferred_element_type=jnp.float32) + xs)
    z = z * gdinv[:, None] + gb_ref[...]
    z = jnp.maximum(z, 0.0)
    y2 = jnp.dot(z, l2w_ref[...], preferred_element_type=jnp.float32)
    y2_ref[...] = y2 + l2b_ref[...]


def _head_body(mhm_ref, y2_ref, l3w_ref, l3b_ref, ow_ref, ob_ref, out_ref):
    def body(k, carry):
        accm, accv = carry
        mbk = jnp.minimum(mhm_ref[pl.ds(k * 128, 128), :], 1.0)
        vbk = jnp.maximum(y2_ref[pl.ds(k * 128, 128), :], 0.0)
        mb16 = mbk.astype(jnp.bfloat16)
        vb16 = vbk.astype(jnp.bfloat16)
        for r in range(16):
            if False:
                mbr = mbk[r * 8:(r + 1) * 8, :]
                vbr = vbk[r * 8:(r + 1) * 8, :]
                sel = lax.dot_general(mbr[:, :, None], vbr[:, None, :],
                                      (((2,), (1,)), ((0,), (0,))),
                                      preferred_element_type=jnp.float32)
                accm = jnp.maximum(accm, jnp.max(sel, axis=0))
            else:
                mbr = mb16[r * 8:(r + 1) * 8, :]
                vbr = vb16[r * 8:(r + 1) * 8, :]
                sel = mbr[:, :, None] * vbr[:, None, :]
                accv = jnp.maximum(accv, jnp.max(sel, axis=0))
        return accm, accv

    accm, accv = lax.fori_loop(
        0, _N // 128, body,
        (jnp.zeros((128, 512), jnp.float32), jnp.zeros((128, 512), jnp.bfloat16)))
    m = jnp.maximum(accm, accv.astype(jnp.float32))
    h = jnp.dot(m, l3w_ref[...], preferred_element_type=jnp.float32)
    h = h + l3b_ref[...]
    o = jnp.dot(h, ow_ref[...], preferred_element_type=jnp.float32)
    out_ref[...] = o + ob_ref[...]


def kernel(edge_index, edge_edge_index, motif_edge_index, node2vec_emb,
           hconv_W, hconv_b, lin1_W, lin1_b, gcn_W, gcn_b,
           lin2_W, lin2_b, eout_W, eout_b, lin3_W, lin3_b, out_W, out_b):
    t01 = _tables_sc_a()(edge_index[0], edge_index[1])
    t23 = _tables_sc_b()(edge_edge_index[0], edge_edge_index[1],
                         motif_edge_index[0], motif_edge_index[1])
    t01_2d = t01.reshape(2 * _N, _N)
    t23_2d = t23.reshape(2 * _N, _N)

    full = lambda r, c: pl.BlockSpec((r, c), lambda i: (0, 0))
    y2 = pl.pallas_call(
        _dense_body,
        grid=(1,),
        in_specs=[
            pl.BlockSpec((_N, _N), lambda i: (0, 0)),
            pl.BlockSpec((_N, _N), lambda i: (1, 0)),
            pl.BlockSpec((_N, _N), lambda i: (0, 0)),
            full(_N, _N), full(_N, _N), full(1, _N), full(_N, _N), full(1, _N),
            full(_N, 512), full(1, 512), full(512, 512), full(1, 512),
        ],
        out_specs=pl.BlockSpec((_N, 512), lambda i: (0, 0)),
        out_shape=jax.ShapeDtypeStruct((_N, 512), jnp.float32),
    )(t01_2d, t01_2d, t23_2d, node2vec_emb, hconv_W, hconv_b.reshape(1, -1),
      lin1_W, lin1_b.reshape(1, -1), gcn_W, gcn_b.reshape(1, -1),
      lin2_W, lin2_b.reshape(1, -1))

    out = pl.pallas_call(
        _head_body,
        grid=(8,),
        in_specs=[
            pl.BlockSpec((_N, 128), lambda i: (1, i)),
            pl.BlockSpec((_N, 512), lambda i: (0, 0)),
            pl.BlockSpec((512, 256), lambda i: (0, 0)),
            pl.BlockSpec((1, 256), lambda i: (0, 0)),
            pl.BlockSpec((256, 128), lambda i: (0, 0)),
            pl.BlockSpec((1, 128), lambda i: (0, 0)),
        ],
        out_specs=pl.BlockSpec((128, 128), lambda i: (i, 0)),
        out_shape=jax.ShapeDtypeStruct((_N, 128), jnp.float32),
    )(t23_2d, y2, lin3_W, lin3_b.reshape(1, -1), out_W, out_b.reshape(1, -1))
    return out
